# baseline (speedup 1.0000x reference)
import functools

import jax
import jax.numpy as jnp
from jax import lax
from jax.experimental import pallas as pl
from jax.experimental.pallas import tpu as pltpu
from jax.experimental.pallas import tpu_sc as plsc

_D = 16
_L = 16
_NC = 2
_NS = 16
_NW = _NC * _NS
_BPW = 512
_G = 16
_NB = _BPW // _G
_W = 128


def _gather(idx, tbl_t):
    mesh = plsc.VectorSubcoreMesh(core_axis_name="c", subcore_axis_name="s")
    n = idx.shape[0]

    @functools.partial(
        pl.kernel,
        out_type=jax.ShapeDtypeStruct((_D, n), jnp.float32),
        mesh=mesh,
        scratch_types=[
            pltpu.VMEM((_BPW,), jnp.int32),
            pltpu.VMEM((3, _G, _D, _W), jnp.float32),
            pltpu.VMEM((_D, _BPW), jnp.float32),
            pltpu.SemaphoreType.DMA,
            pltpu.SemaphoreType.DMA,
            pltpu.SemaphoreType.DMA,
        ],
        compiler_params=pltpu.CompilerParams(needs_layout_passes=False),
        cost_estimate=pl.CostEstimate(
            flops=0, transcendentals=0, bytes_accessed=135_000_000),
    )
    def body(idx_hbm, tbl_hbm, out_hbm, u_v, col_v, out_v, sem0, sem1, sem2):
        wid = lax.axis_index("s") * _NC + lax.axis_index("c")
        pltpu.sync_copy(idx_hbm.at[pl.ds(wid * _BPW, _BPW)], u_v)
        iota = lax.iota(jnp.int32, _L)

        def fire(b, buf, sem):
            uvec = u_v[pl.ds(b * _G, _G)]
            copies = []
            for jj in range(_G):
                c0 = pl.multiple_of((uvec[jj] // _W) * _W, _W)
                copies.append(pltpu.async_copy(
                    tbl_hbm.at[iota, pl.ds(c0, _W)], col_v.at[buf, jj], sem))
            return copies

        def drain(b, buf, copies):
            lanes = u_v[pl.ds(b * _G, _G)] % _W
            bv = jnp.full((_L,), buf, jnp.int32)
            for c in copies:
                c.wait()
            for d in range(_D):
                vals = plsc.load_gather(
                    col_v, [bv, iota, jnp.full((_L,), d, jnp.int32), lanes])
                out_v[d, pl.ds(b * _G, _G)] = vals

        sems = (sem0, sem1, sem2)

        def waits(buf, sem):
            return [pltpu.make_async_copy(
                tbl_hbm.at[iota, pl.ds(0, _W)], col_v.at[buf, jj], sem)
                for jj in range(_G)]

        def pipelined(k, carry):
            b = 3 * k
            fire(b + 2, 2, sem2)
            drain(b, 0, waits(0, sem0))

            @pl.when(b + 3 < _NB)
            def _():
                fire(b + 3, 0, sem0)

            drain(b + 1, 1, waits(1, sem1))

            @pl.when(b + 4 < _NB)
            def _():
                fire(b + 4, 1, sem1)

            drain(b + 2, 2, waits(2, sem2))
            return carry

        fire(0, 0, sem0)
        fire(1, 1, sem1)
        lax.fori_loop(0, _NB // 3, pipelined, 0)
        drain(_NB - 2, 0, waits(0, sem0))
        drain(_NB - 1, 1, waits(1, sem1))
        pltpu.sync_copy(out_v, out_hbm.at[:, pl.ds(wid * _BPW, _BPW)])

    return body(idx, tbl_t)


def kernel(user, user_table, item_table):
    rows_t = _gather(user, user_table.T)
    return (rows_t.T, item_table)


# speedup vs baseline: 1.0418x; 1.0418x over previous
import functools

import jax
import jax.numpy as jnp
from jax import lax
from jax.experimental import pallas as pl
from jax.experimental.pallas import tpu as pltpu
from jax.experimental.pallas import tpu_sc as plsc

_D = 16
_L = 16
_NC = 2
_NS = 16
_NW = _NC * _NS
_BPW = 512
_G = 16
_NB = _BPW // _G
_W = 128


def _gather(idx, tbl_t):
    mesh = plsc.VectorSubcoreMesh(core_axis_name="c", subcore_axis_name="s")
    n = idx.shape[0]

    @functools.partial(
        pl.kernel,
        out_type=jax.ShapeDtypeStruct((_D, n), jnp.float32),
        mesh=mesh,
        scratch_types=[
            pltpu.VMEM((_BPW,), jnp.int32),
            pltpu.VMEM((2, _G, _D, _W), jnp.float32),
            pltpu.VMEM((_D, _BPW), jnp.float32),
            pltpu.SemaphoreType.DMA,
            pltpu.SemaphoreType.DMA,
        ],
        compiler_params=pltpu.CompilerParams(needs_layout_passes=False),
        cost_estimate=pl.CostEstimate(
            flops=0, transcendentals=0, bytes_accessed=135_000_000),
    )
    def body(idx_hbm, tbl_hbm, out_hbm, u_v, col_v, out_v, sem0, sem1):
        wid = lax.axis_index("s") * _NC + lax.axis_index("c")
        pltpu.sync_copy(idx_hbm.at[pl.ds(wid * _BPW, _BPW)], u_v)
        iota = lax.iota(jnp.int32, _L)

        def fire(b, buf, sem):
            uvec = u_v[pl.ds(b * _G, _G)]
            copies = []
            for jj in range(_G):
                c0 = pl.multiple_of((uvec[jj] // _W) * _W, _W)
                copies.append(pltpu.async_copy(
                    tbl_hbm.at[iota, pl.ds(c0, _W)], col_v.at[buf, jj], sem))
            return copies

        def drain(b, buf, copies):
            lanes = u_v[pl.ds(b * _G, _G)] % _W
            bv = jnp.full((_L,), buf, jnp.int32)
            for c in copies:
                c.wait()
            for d in range(_D):
                vals = plsc.load_gather(
                    col_v, [bv, iota, jnp.full((_L,), d, jnp.int32), lanes])
                out_v[d, pl.ds(b * _G, _G)] = vals

        def pipelined(k, carry):
            b0, b1 = 2 * k, 2 * k + 1
            c1 = fire(b1, 1, sem1)
            w0 = [pltpu.make_async_copy(
                tbl_hbm.at[iota, pl.ds(0, _W)], col_v.at[0, jj], sem0)
                for jj in range(_G)]
            drain(b0, 0, w0)

            @pl.when(k < _NB // 2 - 1)
            def _():
                fire(2 * k + 2, 0, sem0)

            drain(b1, 1, c1)
            return carry

        fire(0, 0, sem0)
        lax.fori_loop(0, _NB // 2, pipelined, 0)
        pltpu.sync_copy(out_v, out_hbm.at[:, pl.ds(wid * _BPW, _BPW)])

    return body(idx, tbl_t)


def kernel(user, user_table, item_table):
    rows_t = _gather(user, user_table.T)
    return (rows_t.T, item_table)
